# baseline (speedup 1.0000x reference)
import functools

import jax
import jax.numpy as jnp
from jax import lax
from jax.experimental import pallas as pl
from jax.experimental.pallas import tpu as pltpu
from jax.experimental.pallas import tpu_sc as plsc

N = 10000
D = 128
NC = 2
NS = 16
NW = NC * NS
C = 128
N_PAD = 10240
SLAB = N_PAD // NS
ZROWS = 32
WAL = 8
JPAD = 16


def _agg_body(base_cw, extra, sg_sz, win, x_hbm, src_hbm, dst_hbm, out_hbm,
              src_v, dst_v, rows_v, zbuf_v, acc_sh, sem0, sem1, ssem0, ssem1):
    c = lax.axis_index("c")
    s = lax.axis_index("s")
    w = c * NS + s
    base = w * base_cw + jnp.minimum(w, extra)
    tail = jnp.where(w < extra, 1, 0)

    zv = jnp.zeros((16,), jnp.float32)

    def _zrow(i, _):
        for j in range(D // 16):
            zbuf_v[i, pl.ds(j * 16, 16)] = zv
        return 0

    lax.fori_loop(0, ZROWS, _zrow, 0)
    for r in range(SLAB // ZROWS):
        pltpu.sync_copy(zbuf_v, acc_sh.at[pl.ds(s * SLAB + r * ZROWS, ZROWS)])
    plsc.subcore_barrier()

    gsems = (sem0, sem1)
    ssems = (ssem0, ssem1)

    def _fire(chunk, j):
        return pltpu.async_copy(x_hbm.at[src_v.at[chunk]], rows_v.at[j],
                                gsems[j])

    def _drain(j):
        pltpu.make_async_copy(x_hbm.at[src_v.at[0]], rows_v.at[j],
                              gsems[j]).wait()

    def _scatter(chunk, j):
        return pltpu.async_copy(rows_v.at[j], acc_sh.at[dst_v.at[chunk]],
                                ssems[j], add=True)

    def _super(sg, _):
        cb = base + sg * sg_sz
        cb_al = cb // WAL * WAL
        off = cb - cb_al
        pltpu.sync_copy(src_hbm.at[pl.ds(cb_al, win)], src_v)
        pltpu.sync_copy(dst_hbm.at[pl.ds(cb_al, win)], dst_v)
        _fire(off, 0)
        _fire(off + 1, 1)

        def _pair(p, _):
            _drain(0)
            s0 = _scatter(off + 2 * p, 0)
            _drain(1)
            s1 = _scatter(off + 2 * p + 1, 1)
            s0.wait()
            _fire(off + 2 * p + 2, 0)
            s1.wait()
            _fire(off + 2 * p + 3, 1)
            return 0

        lax.fori_loop(0, sg_sz // 2 - 1, _pair, 0)
        _drain(0)
        s0 = _scatter(off + sg_sz - 2, 0)
        _drain(1)
        s1 = _scatter(off + sg_sz - 1, 1)
        s0.wait()
        s1.wait()
        return 0

    nsg = base_cw // sg_sz
    lax.fori_loop(0, nsg, _super, 0)

    def _tail(t, _):
        cb = base + nsg * sg_sz + t
        cb_al = cb // WAL * WAL
        off = cb - cb_al
        pltpu.sync_copy(src_hbm.at[pl.ds(cb_al, WAL)],
                        src_v.at[pl.ds(0, WAL)])
        pltpu.sync_copy(dst_hbm.at[pl.ds(cb_al, WAL)],
                        dst_v.at[pl.ds(0, WAL)])
        _fire(off, 0).wait()
        _scatter(off, 0).wait()
        return 0

    lax.fori_loop(0, tail, _tail, 0)
    plsc.subcore_barrier()

    pltpu.sync_copy(acc_sh.at[pl.ds(s * SLAB, SLAB)],
                    out_hbm.at[c, pl.ds(s * SLAB, SLAB)])


def _aggregate(x, srcp, dstp):
    chunks = srcp.shape[0] - JPAD
    base_cw = chunks // NW
    extra = chunks % NW
    sg_sz = 2
    for d in range(2, min(40, base_cw) + 1, 2):
        if base_cw % d == 0:
            sg_sz = d
    win = (sg_sz + WAL + WAL - 1) // WAL * WAL
    mesh = plsc.VectorSubcoreMesh(core_axis_name="c", subcore_axis_name="s")
    kern = pl.kernel(
        functools.partial(_agg_body, base_cw, extra, sg_sz, win),
        out_type=jax.ShapeDtypeStruct((NC, N_PAD, D), jnp.float32),
        mesh=mesh,
        scratch_types=[
            pltpu.VMEM((win, C), jnp.int32),
            pltpu.VMEM((win, C), jnp.int32),
            pltpu.VMEM((2, C, D), jnp.float32),
            pltpu.VMEM((ZROWS, D), jnp.float32),
            pltpu.VMEM_SHARED((N_PAD, D), jnp.float32),
            pltpu.SemaphoreType.DMA,
            pltpu.SemaphoreType.DMA,
            pltpu.SemaphoreType.DMA,
            pltpu.SemaphoreType.DMA,
        ],
    )
    return kern(x, srcp, dstp)


def _dense1_body(p_ref, w_ref, b_ref, gw_ref, gb_ref, gms_ref, o_ref):
    agg = p_ref[0, :N, :] + p_ref[1, :N, :]
    z = jnp.dot(agg, w_ref[...], preferred_element_type=jnp.float32) + b_ref[...]
    mean = jnp.mean(z, axis=0, keepdims=True)
    cent = z - mean * gms_ref[...]
    var = jnp.mean(cent * cent, axis=0, keepdims=True)
    o_ref[...] = jnp.tanh(gw_ref[...] * cent * lax.rsqrt(var + 1e-5) + gb_ref[...])


def _dense2_body(p_ref, w_ref, b_ref, o_ref):
    agg = p_ref[0, :N, :] + p_ref[1, :N, :]
    z = jnp.dot(agg, w_ref[...], preferred_element_type=jnp.float32) + b_ref[...]
    o_ref[...] = jnp.tanh(z)


def _dense1(partial, w1t, b1, gw, gb, gms):
    return pl.pallas_call(
        _dense1_body,
        out_shape=jax.ShapeDtypeStruct((N, D), jnp.float32),
    )(partial, w1t, b1, gw, gb, gms)


def _dense2(partial, w2t, b2):
    return pl.pallas_call(
        _dense2_body,
        out_shape=jax.ShapeDtypeStruct((N, D), jnp.float32),
    )(partial, w2t, b2)


def kernel(x, edge_index, W1, b1, gn_weight, gn_bias, gn_mean_scale, W2, b2):
    e = edge_index.shape[1]
    assert e % C == 0, "edge count must be a multiple of the chunk size"
    junk = jnp.zeros((JPAD, C), jnp.int32)
    src = jnp.concatenate([edge_index[0].astype(jnp.int32).reshape(e // C, C),
                           junk])
    dst = jnp.concatenate([edge_index[1].astype(jnp.int32).reshape(e // C, C),
                           junk])

    w1t = W1.T
    w2t = W2.T
    b1r = b1.reshape(1, D)
    b2r = b2.reshape(1, D)
    gwr = gn_weight.reshape(1, D)
    gbr = gn_bias.reshape(1, D)
    gmsr = gn_mean_scale.reshape(1, D)

    p1 = _aggregate(x, src, dst)
    t1 = _dense1(p1, w1t, b1r, gwr, gbr, gmsr)
    p2 = _aggregate(t1, src, dst)
    return _dense2(p2, w2t, b2r)


# speedup vs baseline: 1.3201x; 1.3201x over previous
import functools

import jax
import jax.numpy as jnp
from jax import lax
from jax.experimental import pallas as pl
from jax.experimental.pallas import tpu as pltpu
from jax.experimental.pallas import tpu_sc as plsc

N = 10000
D = 128
NC = 2
NS = 16
NW = NC * NS
C = 128
N_PAD = 10240
SLAB = N_PAD // NS
ZROWS = 32
WAL = 8
JPAD = 16


def _agg_body(base_cw, extra, sg_sz, win, x_hbm, ei_hbm, out_hbm,
              idx_v, rows_v, zbuf_v, acc_sh, sem0, sem1):
    c = lax.axis_index("c")
    s = lax.axis_index("s")
    w = c * NS + s
    base = w * base_cw + jnp.minimum(w, extra)
    tail = jnp.where(w < extra, 1, 0)

    zv = jnp.zeros((16,), jnp.float32)

    def _zrow(i, _):
        for j in range(D // 16):
            zbuf_v[i, pl.ds(j * 16, 16)] = zv
        return 0

    lax.fori_loop(0, ZROWS, _zrow, 0)
    for r in range(SLAB // ZROWS):
        pltpu.sync_copy(zbuf_v, acc_sh.at[pl.ds(s * SLAB + r * ZROWS, ZROWS)])
    plsc.subcore_barrier()

    sems = (sem0, sem1)

    def _fire(chunk, j):
        return pltpu.async_copy(x_hbm.at[idx_v.at[chunk, 0]], rows_v.at[j],
                                sems[j])

    def _drain(j):
        pltpu.make_async_copy(x_hbm.at[idx_v.at[0, 0]], rows_v.at[j],
                              sems[j]).wait()

    def _scatter(chunk, j):
        pltpu.sync_copy(rows_v.at[j], acc_sh.at[idx_v.at[chunk, 1]], add=True)

    def _super(sg, _):
        cb = base + sg * sg_sz
        cb_al = cb // WAL * WAL
        off = cb - cb_al
        pltpu.sync_copy(ei_hbm.at[pl.ds(cb_al, win)], idx_v)
        _fire(off, 0)
        _fire(off + 1, 1)

        def _pair(p, _):
            _drain(0)
            _scatter(off + 2 * p, 0)
            _fire(off + 2 * p + 2, 0)
            _drain(1)
            _scatter(off + 2 * p + 1, 1)
            _fire(off + 2 * p + 3, 1)
            return 0

        lax.fori_loop(0, sg_sz // 2 - 1, _pair, 0)
        _drain(0)
        _scatter(off + sg_sz - 2, 0)
        _drain(1)
        _scatter(off + sg_sz - 1, 1)
        return 0

    nsg = base_cw // sg_sz
    lax.fori_loop(0, nsg, _super, 0)

    def _tail(t, _):
        cb = base + nsg * sg_sz + t
        cb_al = cb // WAL * WAL
        off = cb - cb_al
        pltpu.sync_copy(ei_hbm.at[pl.ds(cb_al, WAL)],
                        idx_v.at[pl.ds(0, WAL)])
        _fire(off, 0).wait()
        _scatter(off, 0)
        return 0

    lax.fori_loop(0, tail, _tail, 0)
    plsc.subcore_barrier()

    pltpu.sync_copy(acc_sh.at[pl.ds(s * SLAB, SLAB)],
                    out_hbm.at[c, pl.ds(s * SLAB, SLAB)])


def _aggregate(x, ei3):
    chunks = ei3.shape[0] - JPAD
    base_cw = chunks // NW
    extra = chunks % NW
    sg_sz = 2
    for d in range(2, min(40, base_cw) + 1, 2):
        if base_cw % d == 0:
            sg_sz = d
    win = (sg_sz + WAL + WAL - 1) // WAL * WAL
    mesh = plsc.VectorSubcoreMesh(core_axis_name="c", subcore_axis_name="s")
    kern = pl.kernel(
        functools.partial(_agg_body, base_cw, extra, sg_sz, win),
        out_type=jax.ShapeDtypeStruct((NC, N_PAD, D), jnp.float32),
        mesh=mesh,
        scratch_types=[
            pltpu.VMEM((win, 2, C), jnp.int32),
            pltpu.VMEM((2, C, D), jnp.float32),
            pltpu.VMEM((ZROWS, D), jnp.float32),
            pltpu.VMEM_SHARED((N_PAD, D), jnp.float32),
            pltpu.SemaphoreType.DMA,
            pltpu.SemaphoreType.DMA,
        ],
    )
    return kern(x, ei3)


def _dense1_body(p_ref, w_ref, b_ref, gw_ref, gb_ref, gms_ref, o_ref):
    agg = p_ref[0, :N, :] + p_ref[1, :N, :]
    z = jnp.dot(agg, w_ref[...], preferred_element_type=jnp.float32) + b_ref[...]
    mean = jnp.mean(z, axis=0, keepdims=True)
    cent = z - mean * gms_ref[...]
    var = jnp.mean(cent * cent, axis=0, keepdims=True)
    o_ref[...] = jnp.tanh(gw_ref[...] * cent * lax.rsqrt(var + 1e-5) + gb_ref[...])


def _dense2_body(p_ref, w_ref, b_ref, o_ref):
    agg = p_ref[0, :N, :] + p_ref[1, :N, :]
    z = jnp.dot(agg, w_ref[...], preferred_element_type=jnp.float32) + b_ref[...]
    o_ref[...] = jnp.tanh(z)


def _dense1(partial, w1t, b1, gw, gb, gms):
    return pl.pallas_call(
        _dense1_body,
        out_shape=jax.ShapeDtypeStruct((N, D), jnp.float32),
    )(partial, w1t, b1, gw, gb, gms)


def _dense2(partial, w2t, b2):
    return pl.pallas_call(
        _dense2_body,
        out_shape=jax.ShapeDtypeStruct((N, D), jnp.float32),
    )(partial, w2t, b2)


def kernel(x, edge_index, W1, b1, gn_weight, gn_bias, gn_mean_scale, W2, b2):
    e = edge_index.shape[1]
    assert e % C == 0, "edge count must be a multiple of the chunk size"
    ei3 = edge_index.astype(jnp.int32).reshape(2, e // C, C).transpose(1, 0, 2)
    ei3 = jnp.concatenate([ei3, jnp.zeros((JPAD, 2, C), jnp.int32)])

    w1t = W1.T
    w2t = W2.T
    b1r = b1.reshape(1, D)
    b2r = b2.reshape(1, D)
    gwr = gn_weight.reshape(1, D)
    gbr = gn_bias.reshape(1, D)
    gmsr = gn_mean_scale.reshape(1, D)

    p1 = _aggregate(x, ei3)
    t1 = _dense1(p1, w1t, b1r, gwr, gbr, gmsr)
    p2 = _aggregate(t1, ei3)
    return _dense2(p2, w2t, b2r)
